# baseline (speedup 1.0000x reference)
import functools
import jax
import jax.numpy as jnp
from jax import lax
from jax.experimental import pallas as pl
from jax.experimental.pallas import tpu as pltpu
from jax.experimental.pallas import tpu_sc as plsc

BATCH = 1024
IN_FEATURES = 128
OUT_FEATURES = 128
GRID_SIZE = 8
MIN_VALUE = -2.0
MAX_VALUE = 2.0

NW = 32
B_PER_W = BATCH // NW
NCHUNK = OUT_FEATURES // 16


def _sc_body(x_hbm, basis_hbm, bias_hbm, out_hbm,
             x_v, bias_v, idx_l, idx_r, w_l, w_r, rows_l, rows_r, out_v, sem):
    wid = lax.axis_index("s") * 2 + lax.axis_index("c")
    base = wid * B_PER_W
    pltpu.sync_copy(x_hbm.at[pl.ds(base, B_PER_W)], x_v)
    pltpu.sync_copy(bias_hbm, bias_v)

    def per_batch(b, _):
        def prep(ii, _):
            xv = x_v[b, pl.ds(ii * 16, 16)]
            scaled = (jnp.clip(xv, MIN_VALUE, MAX_VALUE) - MIN_VALUE) * (
                (GRID_SIZE - 1) / (MAX_VALUE - MIN_VALUE)
            )
            li = jnp.minimum(scaled.astype(jnp.int32), GRID_SIZE - 2)
            rw = scaled - li.astype(jnp.float32)
            i_glob = lax.iota(jnp.int32, 16) + ii * 16
            kl = i_glob * GRID_SIZE + li
            idx_l[pl.ds(ii * 16, 16)] = kl
            idx_r[pl.ds(ii * 16, 16)] = kl + 1
            w_r[pl.ds(ii * 16, 16)] = rw
            w_l[pl.ds(ii * 16, 16)] = 1.0 - rw
            return 0

        lax.fori_loop(0, NCHUNK, prep, 0)

        pltpu.async_copy(basis_hbm.at[idx_l], rows_l, sem).wait()
        pltpu.async_copy(basis_hbm.at[idx_r], rows_r, sem).wait()

        def accum(ii, acc):
            lwv = w_l[pl.ds(ii * 16, 16)]
            rwv = w_r[pl.ds(ii * 16, 16)]
            for j in range(16):
                i = ii * 16 + j
                lw = lwv[j]
                rw = rwv[j]
                acc = tuple(
                    acc[c] + lw * rows_l[i, pl.ds(c * 16, 16)]
                    + rw * rows_r[i, pl.ds(c * 16, 16)]
                    for c in range(NCHUNK)
                )
            return acc

        acc0 = tuple(bias_v[pl.ds(c * 16, 16)] for c in range(NCHUNK))
        acc = lax.fori_loop(0, IN_FEATURES // 16, accum, acc0)
        for c in range(NCHUNK):
            out_v[b, pl.ds(c * 16, 16)] = acc[c]
        return 0

    lax.fori_loop(0, B_PER_W, per_batch, 0)
    pltpu.sync_copy(out_v, out_hbm.at[pl.ds(base, B_PER_W)])


def kernel(inputs, basis, bias):
    basis_rows = basis.reshape(OUT_FEATURES, IN_FEATURES * GRID_SIZE).T
    mesh = plsc.VectorSubcoreMesh(core_axis_name="c", subcore_axis_name="s")
    call = functools.partial(
        pl.kernel,
        mesh=mesh,
        out_type=jax.ShapeDtypeStruct((BATCH, OUT_FEATURES), jnp.float32),
        scratch_types=[
            pltpu.VMEM((B_PER_W, IN_FEATURES), jnp.float32),
            pltpu.VMEM((OUT_FEATURES,), jnp.float32),
            pltpu.VMEM((IN_FEATURES,), jnp.int32),
            pltpu.VMEM((IN_FEATURES,), jnp.int32),
            pltpu.VMEM((IN_FEATURES,), jnp.float32),
            pltpu.VMEM((IN_FEATURES,), jnp.float32),
            pltpu.VMEM((IN_FEATURES, OUT_FEATURES), jnp.float32),
            pltpu.VMEM((IN_FEATURES, OUT_FEATURES), jnp.float32),
            pltpu.VMEM((B_PER_W, OUT_FEATURES), jnp.float32),
            pltpu.SemaphoreType.DMA,
        ],
    )(_sc_body)
    return call(inputs, basis_rows, bias)


# speedup vs baseline: 37.7321x; 37.7321x over previous
import jax
import jax.numpy as jnp
from jax import lax
from jax.experimental import pallas as pl
from jax.experimental.pallas import tpu as pltpu

BATCH = 1024
IN_FEATURES = 128
OUT_FEATURES = 128
GRID_SIZE = 8
MIN_VALUE = -2.0
MAX_VALUE = 2.0

K = IN_FEATURES * GRID_SIZE


def _kan_kernel(x_ref, basis_ref, bias_ref, out_ref):
    x = x_ref[:]
    scaled = (jnp.clip(x, MIN_VALUE, MAX_VALUE) - MIN_VALUE) * (
        (GRID_SIZE - 1) / (MAX_VALUE - MIN_VALUE)
    )
    w = jnp.concatenate(
        [
            jnp.maximum(1.0 - jnp.abs(scaled - float(g)), 0.0).astype(jnp.bfloat16)
            for g in range(GRID_SIZE)
        ],
        axis=1,
    )
    c_idx = lax.broadcasted_iota(jnp.int32, (K, K), 0)
    r_idx = lax.broadcasted_iota(jnp.int32, (K, K), 1)
    perm = (r_idx == ((c_idx & 7) << 7) + (c_idx >> 3)).astype(jnp.bfloat16)
    bflat = basis_ref[:].astype(jnp.bfloat16)
    bg = lax.dot_general(
        bflat, perm, (((1,), (0,)), ((), ())),
        preferred_element_type=jnp.float32,
    ).astype(jnp.bfloat16)
    out = lax.dot_general(
        w, bg, (((1,), (1,)), ((), ())),
        preferred_element_type=jnp.float32,
    )
    out_ref[:] = out + bias_ref[:]


def kernel(inputs, basis, bias):
    basis_flat = basis.reshape(OUT_FEATURES, K)
    bias2d = bias.reshape(1, OUT_FEATURES)
    return pl.pallas_call(
        _kan_kernel,
        grid=(1,),
        in_specs=[
            pl.BlockSpec((BATCH, IN_FEATURES), lambda i: (0, 0)),
            pl.BlockSpec((OUT_FEATURES, K), lambda i: (0, 0)),
            pl.BlockSpec((1, OUT_FEATURES), lambda i: (0, 0)),
        ],
        out_specs=pl.BlockSpec((BATCH, OUT_FEATURES), lambda i: (0, 0)),
        out_shape=jax.ShapeDtypeStruct((BATCH, OUT_FEATURES), jnp.float32),
    )(inputs, basis_flat, bias2d)


# speedup vs baseline: 44.5357x; 1.1803x over previous
import jax
import jax.numpy as jnp
from jax.experimental import pallas as pl
from jax.experimental.pallas import tpu as pltpu

BATCH = 1024
IN_FEATURES = 128
OUT_FEATURES = 128
GRID_SIZE = 8
MIN_VALUE = -2.0
MAX_VALUE = 2.0

BLOCK_B = 1024


def _kan_kernel(x_ref, basis_ref, bias_ref, out_ref):
    x = x_ref[:]
    scaled = (jnp.clip(x, MIN_VALUE, MAX_VALUE) - MIN_VALUE) * (
        (GRID_SIZE - 1) / (MAX_VALUE - MIN_VALUE)
    )
    acc = jnp.broadcast_to(bias_ref[:], out_ref.shape)
    for g in range(GRID_SIZE):
        w = jnp.maximum(1.0 - jnp.abs(scaled - float(g)), 0.0).astype(jnp.bfloat16)
        acc = acc + jnp.dot(w, basis_ref[g], preferred_element_type=jnp.float32)
    out_ref[:] = acc


def kernel(inputs, basis, bias):
    basis_t = jnp.transpose(basis, (2, 1, 0)).astype(jnp.bfloat16)
    bias2d = bias.reshape(1, OUT_FEATURES)
    grid = (BATCH // BLOCK_B,)
    return pl.pallas_call(
        _kan_kernel,
        grid=grid,
        in_specs=[
            pl.BlockSpec((BLOCK_B, IN_FEATURES), lambda i: (i, 0)),
            pl.BlockSpec((GRID_SIZE, IN_FEATURES, OUT_FEATURES), lambda i: (0, 0, 0)),
            pl.BlockSpec((1, OUT_FEATURES), lambda i: (0, 0)),
        ],
        out_specs=pl.BlockSpec((BLOCK_B, OUT_FEATURES), lambda i: (i, 0)),
        out_shape=jax.ShapeDtypeStruct((BATCH, OUT_FEATURES), jnp.float32),
    )(inputs, basis_t, bias2d)
